# Initial kernel scaffold; baseline (speedup 1.0000x reference)
import jax
import jax.numpy as jnp
from jax.experimental import pallas as pl


def kernel(seq_len, rel_pos_bias_table):
    raise NotImplementedError("write your pallas kernel here")


# speedup vs baseline: 40.8625x; 40.8625x over previous
import functools
import math

import jax
import jax.numpy as jnp
from jax import lax
from jax.experimental import pallas as pl
from jax.experimental.pallas import tpu as pltpu
from jax.experimental.pallas import tpu_sc as plsc

N_HEADS = 16
SEQ = 2048
NUM_BUCKETS = 32
WPAD = 4112
NSHIFT = 8
WROW = 4096
NC, NS = 2, 16
ROWS_PER_W = (N_HEADS * SEQ) // (NC * NS)
BATCH = 16


def _bucket_indices():
    num_buckets = NUM_BUCKETS
    max_distance = max(SEQ, 2)
    p = jnp.arange(WPAD, dtype=jnp.int32)
    n = jnp.abs(p - (SEQ - 1))
    max_exact = max(1, num_buckets // 2)
    is_small = n < max_exact
    n_float = jnp.maximum(n.astype(jnp.float32), 1.0)
    log_scale = math.log(max_distance / max_exact) if max_distance > max_exact else 1.0
    log_scale = max(log_scale, 1e-06)
    val_if_large = max_exact + (
        jnp.log(n_float / max_exact) / log_scale * (num_buckets - max_exact)
    ).astype(jnp.int32)
    val_if_large = jnp.clip(val_if_large, max_exact, num_buckets - 1)
    return jnp.where(is_small, n.astype(jnp.int32), val_if_large)


def _sc_body(widx_hbm, table_hbm, out_hbm, widx_v, table_v, wsh_v, sem):
    wid = lax.axis_index("c") * NS + lax.axis_index("s")
    head = wid // 2
    half = wid % 2

    pltpu.sync_copy(widx_hbm, widx_v)
    pltpu.sync_copy(table_hbm, table_v)

    iota = lax.iota(jnp.int32, 16)
    hvec = jnp.full((16,), head, dtype=jnp.int32)

    for r in range(NSHIFT):
        def build(k, _, r=r):
            base = k * 16
            idx = plsc.load_gather(widx_v, [iota + (base + r)])
            vals = plsc.load_gather(table_v, [idx, hvec])
            wsh_v[pl.ds(r * WROW + base, 16)] = vals
            return _
        lax.fori_loop(0, WROW // 16, build, None)

    row0 = half * ROWS_PER_W

    def rows(g, _):
        copies = []
        for t in range(BATCH):
            i = row0 + g * BATCH + t
            o = (SEQ - 1) - i
            r = jnp.bitwise_and(o, NSHIFT - 1)
            off = pl.multiple_of(r * WROW + (o - r), NSHIFT)
            dst = pl.multiple_of((head * SEQ + i) * SEQ, 256)
            cp = pltpu.make_async_copy(
                wsh_v.at[pl.ds(off, SEQ)], out_hbm.at[pl.ds(dst, SEQ)], sem)
            cp.start()
            copies.append(cp)
        for cp in copies:
            cp.wait()
        return _

    lax.fori_loop(0, ROWS_PER_W // BATCH, rows, None)


def kernel(seq_len, rel_pos_bias_table):
    del seq_len
    widx = _bucket_indices()
    mesh = plsc.VectorSubcoreMesh(core_axis_name="c", subcore_axis_name="s")
    run = functools.partial(
        pl.kernel,
        out_type=jax.ShapeDtypeStruct((N_HEADS * SEQ * SEQ,), jnp.float32),
        mesh=mesh,
        compiler_params=pltpu.CompilerParams(needs_layout_passes=False),
        scratch_types=[
            pltpu.VMEM((WPAD,), jnp.int32),
            pltpu.VMEM((NUM_BUCKETS, N_HEADS), jnp.float32),
            pltpu.VMEM((NSHIFT * WROW,), jnp.float32),
            pltpu.SemaphoreType.DMA,
        ],
    )(_sc_body)
    out = run(widx, rel_pos_bias_table)
    return out.reshape(1, N_HEADS, SEQ, SEQ)
